# baseline (speedup 1.0000x reference)
import dataclasses
import functools

import jax
import jax.numpy as jnp
from jax import lax
from jax.experimental import pallas as pl
from jax.experimental.pallas import tpu as pltpu
from jax.experimental.pallas import tpu_sc as plsc

V = 16


@functools.lru_cache(maxsize=None)
def _build(B, N, MAXLEN, D):
    M = N // 2
    NVh = M // V
    NPAIRh = NVh // 2
    CH = 256
    NCH = M // CH
    num_phases = (NVh - 1).bit_length()

    mesh = plsc.VectorSubcoreMesh(core_axis_name="c", subcore_axis_name="s")

    def body(coords_hbm, pe_hbm, out_hbm, cbuf, keys, vals, pkeys, pvals,
             ranks, orow2d, rowbuf, rowbuf2, metabuf,
             skeys, svals, smeta, sem, wsem0, wsem1):
        c = lax.axis_index("c")
        s = lax.axis_index("s")
        u = s // 2
        half = s % 2
        task = c * B + u
        b = task // 2
        ch = task % 2
        lanes = lax.iota(jnp.int32, V)

        pe2 = pe_hbm

        def cross_pass(off_fn, n_pair):
            @plsc.parallel_loop(0, n_pair, unroll=16)
            def _(t):
                lo_off, hi_off = off_fn(t)
                kL = keys[pl.ds(lo_off, V)]
                kH = keys[pl.ds(hi_off, V)]
                vL = vals[pl.ds(lo_off, V)]
                vH = vals[pl.ds(hi_off, V)]
                cmp = kL <= kH
                keys[pl.ds(lo_off, V)] = jnp.where(cmp, kL, kH)
                vals[pl.ds(lo_off, V)] = jnp.where(cmp, vL, vH)
                keys[pl.ds(hi_off, V)] = jnp.where(cmp, kH, kL)
                vals[pl.ds(hi_off, V)] = jnp.where(cmp, vH, vL)

        def split_pass(m):
            @plsc.parallel_loop(0, NPAIRh, unroll=4)
            def _(t):
                p = m // V
                i = lax.rem(t, p) if p > 1 else t * 0
                pair = lax.div(t, p) if p > 1 else t
                a = pair * (2 * m) + V * i
                bb = pair * (2 * m) + 2 * m - V * (i + 1)
                kA = keys[pl.ds(a, V)]
                vA = vals[pl.ds(a, V)]
                kB = lax.rev(keys[pl.ds(bb, V)], (0,))
                vB = lax.rev(vals[pl.ds(bb, V)], (0,))
                cmp = kA <= kB
                keys[pl.ds(a, V)] = jnp.where(cmp, kA, kB)
                vals[pl.ds(a, V)] = jnp.where(cmp, vA, vB)
                keys[pl.ds(bb, V)] = lax.rev(jnp.where(cmp, kB, kA), (0,))
                vals[pl.ds(bb, V)] = lax.rev(jnp.where(cmp, vB, vA), (0,))

        def stage_pass(d):
            q = (d // V).bit_length() - 1

            def off(t):
                i = lax.rem(t, 1 << q) if q > 0 else t * 0
                blk = lax.div(t, 1 << q) if q > 0 else t
                lo = blk * (2 * d) + V * i
                return lo, lo + d

            cross_pass(off, NPAIRh)

        def vsort_pass():
            @plsc.parallel_loop(0, NVh, unroll=8)
            def _(j):
                k = keys[pl.ds(j * V, V)]
                v = vals[pl.ds(j * V, V)]
                ks, vs = plsc.sort_key_val(k, v)
                keys[pl.ds(j * V, V)] = ks
                vals[pl.ds(j * V, V)] = vs

        @pl.when(u < B)
        def _():
            pltpu.sync_copy(
                coords_hbm.at[b, pl.ds(half * (M // 128), M // 128)], cbuf)

            @plsc.parallel_loop(0, NVh, unroll=2)
            def _(j):
                row = j * V + lanes
                k = plsc.load_gather(
                    cbuf, [row >> 7, jnp.full((V,), ch, jnp.int32), row & 127])
                ks, vs = plsc.sort_key_val(k, row + half * M)
                keys[pl.ds(j * V, V)] = ks
                vals[pl.ds(j * V, V)] = vs

            for p in range(num_phases):
                m = V << p
                split_pass(m)
                for qq in range(p - 1, -1, -1):
                    stage_pass(V << qq)
                vsort_pass()

            cp1 = pltpu.async_copy(keys, skeys.at[s], sem)
            cp2 = pltpu.async_copy(vals, svals.at[s], sem)
            cp1.wait()
            cp2.wait()

        plsc.subcore_barrier()

        @pl.when(u < B)
        def _():
            cp1 = pltpu.async_copy(skeys.at[s ^ 1], pkeys, sem)
            cp2 = pltpu.async_copy(svals.at[s ^ 1], pvals, sem)
            cp1.wait()
            cp2.wait()
            hm = jnp.full((V,), half, jnp.int32) == 0

            @plsc.parallel_loop(0, NVh, unroll=2)
            def _(i):
                mk = keys[pl.ds(i * V, V)]
                mv = vals[pl.ds(i * V, V)]
                pk = lax.rev(pkeys[pl.ds(M - V - V * i, V)], (0,))
                pv = lax.rev(pvals[pl.ds(M - V - V * i, V)], (0,))
                c0 = mk <= pk
                c1 = pk <= mk
                cmp = jnp.where(hm, c0, c1)
                keys[pl.ds(i * V, V)] = jnp.where(cmp, mk, pk)
                vals[pl.ds(i * V, V)] = jnp.where(cmp, mv, pv)

            for qq in range(num_phases - 1, -1, -1):
                stage_pass(V << qq)
            vsort_pass()

            obase = b * 2 * N + ch * 8

            @plsc.parallel_loop(0, NVh, unroll=2, carry=jnp.int32(0))
            def tme(j, carry):
                cur = keys[pl.ds(j * V, V)]
                prev_idx = jnp.maximum(j * V - 1 + lanes, 0)
                prevs = plsc.load_gather(keys, [prev_idx])
                first = jnp.logical_and(j == 0, lanes == 0)
                flag = jnp.where(jnp.logical_or(cur != prevs, first), 1, 0)
                csum = plsc.cumsum(flag)
                ranks[pl.ds(j * V, V)] = csum + carry - 1
                pos = vals[pl.ds(j * V, V)]
                orow2d[lax.div(j, CH // V),
                       pl.ds(lax.rem(j, CH // V) * V, V)] = (
                    obase + ((pos >> 3) << 4) + (pos & 7))
                return carry + jnp.max(csum)

            @pl.when(half == 0)
            def _():
                lastk = plsc.bitcast(keys[pl.ds(M - V, V)], jnp.int32)
                lk = jnp.sum(jnp.where(lanes == V - 1, lastk, 0))
                metabuf[...] = jnp.where(
                    lanes == 0, tme, jnp.where(lanes == 1, lk, 0))
                pltpu.sync_copy(metabuf, smeta.at[s])

        plsc.subcore_barrier()

        @pl.when(jnp.logical_and(u < B, half == 1))
        def _():
            pltpu.sync_copy(smeta.at[s - 1], metabuf)
            mv = metabuf[...]
            t0 = jnp.sum(jnp.where(lanes == 0, mv, 0))
            lk = jnp.sum(jnp.where(lanes == 1, mv, 0))
            k0 = plsc.bitcast(keys[pl.ds(0, V)], jnp.int32)
            fk = jnp.sum(jnp.where(lanes == 0, k0, 0))
            adj = t0 + jnp.where(fk == lk, -1, 0)

            @plsc.parallel_loop(0, NVh, unroll=2)
            def _(j):
                ranks[pl.ds(j * V, V)] = ranks[pl.ds(j * V, V)] + adj

        @pl.when(u < B)
        def _():
            bufs = (rowbuf, rowbuf2)
            wsems = (wsem0, wsem1)

            @pl.loop(0, NCH, step=2)
            def _(t):
                for par in (0, 1):
                    tt = t + par
                    buf = bufs[par]
                    wsem = wsems[par]

                    @pl.when(tt >= 2)
                    def _():
                        pltpu.make_async_copy(
                            buf, out_hbm.at[orow2d.at[par]], wsem).wait()

                    pltpu.async_copy(
                        pe2.at[ranks.at[pl.ds(tt * CH, CH)]], buf, sem
                    ).wait()
                    pltpu.async_copy(buf, out_hbm.at[orow2d.at[tt]], wsem)

            for par in (0, 1):
                pltpu.make_async_copy(
                    bufs[par], out_hbm.at[orow2d.at[par]], wsems[par]).wait()

    cp = pltpu.CompilerParams()
    if "needs_layout_passes" in pltpu.CompilerParams.__dataclass_fields__:
        cp = dataclasses.replace(cp, needs_layout_passes=False)
    if "use_tc_tiling_on_sc" in pltpu.CompilerParams.__dataclass_fields__:
        cp = dataclasses.replace(cp, use_tc_tiling_on_sc=False)

    kern = pl.kernel(
        body,
        compiler_params=cp,
        out_type=jax.ShapeDtypeStruct((B * 2 * N, D), jnp.float32),
        mesh=mesh,
        scratch_types=[
            pltpu.VMEM((M // 128, 2, 128), jnp.float32),
            pltpu.VMEM((M,), jnp.float32),
            pltpu.VMEM((M,), jnp.int32),
            pltpu.VMEM((M,), jnp.float32),
            pltpu.VMEM((M,), jnp.int32),
            pltpu.VMEM((M,), jnp.int32),
            pltpu.VMEM((NCH, CH), jnp.int32),
            pltpu.VMEM((CH, D), jnp.float32),
            pltpu.VMEM((CH, D), jnp.float32),
            pltpu.VMEM((V,), jnp.int32),
            pltpu.VMEM_SHARED((2 * B, M), jnp.float32),
            pltpu.VMEM_SHARED((2 * B, M), jnp.int32),
            pltpu.VMEM_SHARED((2 * B, V), jnp.int32),
            pltpu.SemaphoreType.DMA,
            pltpu.SemaphoreType.DMA,
            pltpu.SemaphoreType.DMA,
        ],
    )
    return kern


def kernel(coords, pe):
    B, N, _ = coords.shape
    MAXLEN, D = pe.shape
    ct = coords.reshape(B, N // 128, 128, 2).transpose(0, 1, 3, 2)
    out2d = _build(B, N, MAXLEN, D)(ct, pe)
    out5d = out2d.reshape(B, N // 8, 2, 8, D)
    return out5d.transpose(0, 1, 3, 2, 4).reshape(B, N, 2 * D)


# speedup vs baseline: 1.0434x; 1.0434x over previous
import dataclasses
import functools

import jax
import jax.numpy as jnp
from jax import lax
from jax.experimental import pallas as pl
from jax.experimental.pallas import tpu as pltpu
from jax.experimental.pallas import tpu_sc as plsc

V = 16


@functools.lru_cache(maxsize=None)
def _build(B, N, MAXLEN, D):
    M = N // 2
    NVh = M // V
    NPAIRh = NVh // 2
    CH = 256
    NCH = M // CH
    num_phases = (NVh - 1).bit_length()

    mesh = plsc.VectorSubcoreMesh(core_axis_name="c", subcore_axis_name="s")

    def body(coords_hbm, pe_hbm, out_hbm, cbuf, keys, vals, pkeys, pvals,
             ranks, orow2d, rowbuf, rowbuf2, metabuf,
             skeys, svals, smeta, sem, wsem0, wsem1):
        c = lax.axis_index("c")
        s = lax.axis_index("s")
        u = s // 2
        half = s % 2
        task = c * B + u
        b = task // 2
        ch = task % 2
        lanes = lax.iota(jnp.int32, V)

        pe2 = pe_hbm

        def cross_pass(off_fn, n_pair):
            @plsc.parallel_loop(0, n_pair, unroll=8)
            def _(t):
                lo_off, hi_off = off_fn(t)
                kL = keys[pl.ds(lo_off, V)]
                kH = keys[pl.ds(hi_off, V)]
                vL = vals[pl.ds(lo_off, V)]
                vH = vals[pl.ds(hi_off, V)]
                cmp = kL <= kH
                keys[pl.ds(lo_off, V)] = jnp.where(cmp, kL, kH)
                vals[pl.ds(lo_off, V)] = jnp.where(cmp, vL, vH)
                keys[pl.ds(hi_off, V)] = jnp.where(cmp, kH, kL)
                vals[pl.ds(hi_off, V)] = jnp.where(cmp, vH, vL)

        def split_pass(m):
            @plsc.parallel_loop(0, NPAIRh, unroll=4)
            def _(t):
                p = m // V
                i = lax.rem(t, p) if p > 1 else t * 0
                pair = lax.div(t, p) if p > 1 else t
                a = pair * (2 * m) + V * i
                bb = pair * (2 * m) + 2 * m - V * (i + 1)
                kA = keys[pl.ds(a, V)]
                vA = vals[pl.ds(a, V)]
                kB = lax.rev(keys[pl.ds(bb, V)], (0,))
                vB = lax.rev(vals[pl.ds(bb, V)], (0,))
                cmp = kA <= kB
                keys[pl.ds(a, V)] = jnp.where(cmp, kA, kB)
                vals[pl.ds(a, V)] = jnp.where(cmp, vA, vB)
                keys[pl.ds(bb, V)] = lax.rev(jnp.where(cmp, kB, kA), (0,))
                vals[pl.ds(bb, V)] = lax.rev(jnp.where(cmp, vB, vA), (0,))

        def stage_pass(d):
            q = (d // V).bit_length() - 1

            def off(t):
                i = lax.rem(t, 1 << q) if q > 0 else t * 0
                blk = lax.div(t, 1 << q) if q > 0 else t
                lo = blk * (2 * d) + V * i
                return lo, lo + d

            cross_pass(off, NPAIRh)

        def vsort_pass():
            @plsc.parallel_loop(0, NVh, unroll=4)
            def _(j):
                k = keys[pl.ds(j * V, V)]
                v = vals[pl.ds(j * V, V)]
                ks, vs = plsc.sort_key_val(k, v)
                keys[pl.ds(j * V, V)] = ks
                vals[pl.ds(j * V, V)] = vs

        @pl.when(u < B)
        def _():
            pltpu.sync_copy(
                coords_hbm.at[b, pl.ds(half * (M // 128), M // 128)], cbuf)

            @plsc.parallel_loop(0, NVh, unroll=4)
            def _(j):
                row = j * V + lanes
                k = plsc.load_gather(
                    cbuf, [row >> 7, jnp.full((V,), ch, jnp.int32), row & 127])
                ks, vs = plsc.sort_key_val(k, row + half * M)
                keys[pl.ds(j * V, V)] = ks
                vals[pl.ds(j * V, V)] = vs

            for p in range(num_phases):
                m = V << p
                split_pass(m)
                for qq in range(p - 1, -1, -1):
                    stage_pass(V << qq)
                vsort_pass()

            cp1 = pltpu.async_copy(keys, skeys.at[s], sem)
            cp2 = pltpu.async_copy(vals, svals.at[s], sem)
            cp1.wait()
            cp2.wait()

        plsc.subcore_barrier()

        @pl.when(u < B)
        def _():
            cp1 = pltpu.async_copy(skeys.at[s ^ 1], pkeys, sem)
            cp2 = pltpu.async_copy(svals.at[s ^ 1], pvals, sem)
            cp1.wait()
            cp2.wait()
            hm = jnp.full((V,), half, jnp.int32) == 0

            @plsc.parallel_loop(0, NVh, unroll=4)
            def _(i):
                mk = keys[pl.ds(i * V, V)]
                mv = vals[pl.ds(i * V, V)]
                pk = lax.rev(pkeys[pl.ds(M - V - V * i, V)], (0,))
                pv = lax.rev(pvals[pl.ds(M - V - V * i, V)], (0,))
                c0 = mk <= pk
                c1 = pk <= mk
                cmp = jnp.where(hm, c0, c1)
                keys[pl.ds(i * V, V)] = jnp.where(cmp, mk, pk)
                vals[pl.ds(i * V, V)] = jnp.where(cmp, mv, pv)

            for qq in range(num_phases - 1, -1, -1):
                stage_pass(V << qq)
            vsort_pass()

            obase = b * 2 * N + ch * 8

            @plsc.parallel_loop(0, NVh, unroll=2, carry=jnp.int32(0))
            def tme(j, carry):
                cur = keys[pl.ds(j * V, V)]
                prev_idx = jnp.maximum(j * V - 1 + lanes, 0)
                prevs = plsc.load_gather(keys, [prev_idx])
                first = jnp.logical_and(j == 0, lanes == 0)
                flag = jnp.where(jnp.logical_or(cur != prevs, first), 1, 0)
                csum = plsc.cumsum(flag)
                ranks[pl.ds(j * V, V)] = csum + carry - 1
                pos = vals[pl.ds(j * V, V)]
                orow2d[lax.div(j, CH // V),
                       pl.ds(lax.rem(j, CH // V) * V, V)] = (
                    obase + ((pos >> 3) << 4) + (pos & 7))
                return carry + jnp.max(csum)

            @pl.when(half == 0)
            def _():
                lastk = plsc.bitcast(keys[pl.ds(M - V, V)], jnp.int32)
                lk = jnp.sum(jnp.where(lanes == V - 1, lastk, 0))
                metabuf[...] = jnp.where(
                    lanes == 0, tme, jnp.where(lanes == 1, lk, 0))
                pltpu.sync_copy(metabuf, smeta.at[s])

        plsc.subcore_barrier()

        @pl.when(jnp.logical_and(u < B, half == 1))
        def _():
            pltpu.sync_copy(smeta.at[s - 1], metabuf)
            mv = metabuf[...]
            t0 = jnp.sum(jnp.where(lanes == 0, mv, 0))
            lk = jnp.sum(jnp.where(lanes == 1, mv, 0))
            k0 = plsc.bitcast(keys[pl.ds(0, V)], jnp.int32)
            fk = jnp.sum(jnp.where(lanes == 0, k0, 0))
            adj = t0 + jnp.where(fk == lk, -1, 0)

            @plsc.parallel_loop(0, NVh, unroll=2)
            def _(j):
                ranks[pl.ds(j * V, V)] = ranks[pl.ds(j * V, V)] + adj

        @pl.when(u < B)
        def _():
            bufs = (rowbuf, rowbuf2)
            wsems = (wsem0, wsem1)

            @pl.loop(0, NCH, step=2)
            def _(t):
                for par in (0, 1):
                    tt = t + par
                    buf = bufs[par]
                    wsem = wsems[par]

                    @pl.when(tt >= 2)
                    def _():
                        pltpu.make_async_copy(
                            buf, out_hbm.at[orow2d.at[par]], wsem).wait()

                    pltpu.async_copy(
                        pe2.at[ranks.at[pl.ds(tt * CH, CH)]], buf, sem
                    ).wait()
                    pltpu.async_copy(buf, out_hbm.at[orow2d.at[tt]], wsem)

            for par in (0, 1):
                pltpu.make_async_copy(
                    bufs[par], out_hbm.at[orow2d.at[par]], wsems[par]).wait()

    cp = pltpu.CompilerParams()
    if "needs_layout_passes" in pltpu.CompilerParams.__dataclass_fields__:
        cp = dataclasses.replace(cp, needs_layout_passes=False)
    if "use_tc_tiling_on_sc" in pltpu.CompilerParams.__dataclass_fields__:
        cp = dataclasses.replace(cp, use_tc_tiling_on_sc=False)

    kern = pl.kernel(
        body,
        compiler_params=cp,
        out_type=jax.ShapeDtypeStruct((B * 2 * N, D), jnp.float32),
        mesh=mesh,
        scratch_types=[
            pltpu.VMEM((M // 128, 2, 128), jnp.float32),
            pltpu.VMEM((M,), jnp.float32),
            pltpu.VMEM((M,), jnp.int32),
            pltpu.VMEM((M,), jnp.float32),
            pltpu.VMEM((M,), jnp.int32),
            pltpu.VMEM((M,), jnp.int32),
            pltpu.VMEM((NCH, CH), jnp.int32),
            pltpu.VMEM((CH, D), jnp.float32),
            pltpu.VMEM((CH, D), jnp.float32),
            pltpu.VMEM((V,), jnp.int32),
            pltpu.VMEM_SHARED((2 * B, M), jnp.float32),
            pltpu.VMEM_SHARED((2 * B, M), jnp.int32),
            pltpu.VMEM_SHARED((2 * B, V), jnp.int32),
            pltpu.SemaphoreType.DMA,
            pltpu.SemaphoreType.DMA,
            pltpu.SemaphoreType.DMA,
        ],
    )
    return kern


def kernel(coords, pe):
    B, N, _ = coords.shape
    MAXLEN, D = pe.shape
    ct = coords.reshape(B, N // 128, 128, 2).transpose(0, 1, 3, 2)
    out2d = _build(B, N, MAXLEN, D)(ct, pe)
    out5d = out2d.reshape(B, N // 8, 2, 8, D)
    return out5d.transpose(0, 1, 3, 2, 4).reshape(B, N, 2 * D)


# speedup vs baseline: 1.4748x; 1.4135x over previous
import dataclasses
import functools

import jax
import jax.numpy as jnp
from jax import lax
from jax.experimental import pallas as pl
from jax.experimental.pallas import tpu as pltpu
from jax.experimental.pallas import tpu_sc as plsc

V = 16


@functools.lru_cache(maxsize=None)
def _build(B, N, MAXLEN, D):
    M = N // 2
    NVh = M // V
    NPAIRh = NVh // 2
    CH = 256
    NCH = M // CH
    num_phases = (NVh - 1).bit_length()

    mesh = plsc.VectorSubcoreMesh(core_axis_name="c", subcore_axis_name="s")

    def body(coords_hbm, pe_hbm, out_hbm, cbuf, keys, vals, pkeys, pvals,
             ranks, orow2d, rowbuf, rowbuf2, metabuf,
             skeys, svals, smeta, spe, sem, wsem0, wsem1, psem):
        c = lax.axis_index("c")
        s = lax.axis_index("s")
        u = s // 2
        half = s % 2
        task = c * B + u
        b = task // 2
        ch = task % 2
        lanes = lax.iota(jnp.int32, V)

        PS = N // (2 * B)
        cpe = pltpu.async_copy(
            pe_hbm.at[pl.ds(s * PS, PS)], spe.at[pl.ds(s * PS, PS)], psem)

        def cross_pass(off_fn, n_pair):
            @plsc.parallel_loop(0, n_pair, unroll=8)
            def _(t):
                lo_off, hi_off = off_fn(t)
                kL = keys[pl.ds(lo_off, V)]
                kH = keys[pl.ds(hi_off, V)]
                vL = vals[pl.ds(lo_off, V)]
                vH = vals[pl.ds(hi_off, V)]
                cmp = kL <= kH
                keys[pl.ds(lo_off, V)] = jnp.where(cmp, kL, kH)
                vals[pl.ds(lo_off, V)] = jnp.where(cmp, vL, vH)
                keys[pl.ds(hi_off, V)] = jnp.where(cmp, kH, kL)
                vals[pl.ds(hi_off, V)] = jnp.where(cmp, vH, vL)

        def split_pass(m):
            @plsc.parallel_loop(0, NPAIRh, unroll=4)
            def _(t):
                p = m // V
                i = lax.rem(t, p) if p > 1 else t * 0
                pair = lax.div(t, p) if p > 1 else t
                a = pair * (2 * m) + V * i
                bb = pair * (2 * m) + 2 * m - V * (i + 1)
                kA = keys[pl.ds(a, V)]
                vA = vals[pl.ds(a, V)]
                kB = lax.rev(keys[pl.ds(bb, V)], (0,))
                vB = lax.rev(vals[pl.ds(bb, V)], (0,))
                cmp = kA <= kB
                keys[pl.ds(a, V)] = jnp.where(cmp, kA, kB)
                vals[pl.ds(a, V)] = jnp.where(cmp, vA, vB)
                keys[pl.ds(bb, V)] = lax.rev(jnp.where(cmp, kB, kA), (0,))
                vals[pl.ds(bb, V)] = lax.rev(jnp.where(cmp, vB, vA), (0,))

        def stage_pass(d):
            q = (d // V).bit_length() - 1

            def off(t):
                i = lax.rem(t, 1 << q) if q > 0 else t * 0
                blk = lax.div(t, 1 << q) if q > 0 else t
                lo = blk * (2 * d) + V * i
                return lo, lo + d

            cross_pass(off, NPAIRh)

        def vsort_pass():
            @plsc.parallel_loop(0, NVh, unroll=4)
            def _(j):
                k = keys[pl.ds(j * V, V)]
                v = vals[pl.ds(j * V, V)]
                ks, vs = plsc.sort_key_val(k, v)
                keys[pl.ds(j * V, V)] = ks
                vals[pl.ds(j * V, V)] = vs

        @pl.when(u < B)
        def _():
            pltpu.sync_copy(
                coords_hbm.at[b, pl.ds(half * (M // 128), M // 128)], cbuf)

            @plsc.parallel_loop(0, NVh, unroll=4)
            def _(j):
                row = j * V + lanes
                k = plsc.load_gather(
                    cbuf, [row >> 7, jnp.full((V,), ch, jnp.int32), row & 127])
                ks, vs = plsc.sort_key_val(k, row + half * M)
                keys[pl.ds(j * V, V)] = ks
                vals[pl.ds(j * V, V)] = vs

            for p in range(num_phases):
                m = V << p
                split_pass(m)
                for qq in range(p - 1, -1, -1):
                    stage_pass(V << qq)
                vsort_pass()

            cp1 = pltpu.async_copy(keys, skeys.at[s], sem)
            cp2 = pltpu.async_copy(vals, svals.at[s], sem)
            cp1.wait()
            cp2.wait()

        plsc.subcore_barrier()

        @pl.when(u < B)
        def _():
            cp1 = pltpu.async_copy(skeys.at[s ^ 1], pkeys, sem)
            cp2 = pltpu.async_copy(svals.at[s ^ 1], pvals, sem)
            cp1.wait()
            cp2.wait()
            hm = jnp.full((V,), half, jnp.int32) == 0

            @plsc.parallel_loop(0, NVh, unroll=4)
            def _(i):
                mk = keys[pl.ds(i * V, V)]
                mv = vals[pl.ds(i * V, V)]
                pk = lax.rev(pkeys[pl.ds(M - V - V * i, V)], (0,))
                pv = lax.rev(pvals[pl.ds(M - V - V * i, V)], (0,))
                c0 = mk <= pk
                c1 = pk <= mk
                cmp = jnp.where(hm, c0, c1)
                keys[pl.ds(i * V, V)] = jnp.where(cmp, mk, pk)
                vals[pl.ds(i * V, V)] = jnp.where(cmp, mv, pv)

            for qq in range(num_phases - 1, -1, -1):
                stage_pass(V << qq)
            vsort_pass()

            obase = b * 2 * N + ch * 8

            @plsc.parallel_loop(0, NVh, unroll=2, carry=jnp.int32(0))
            def tme(j, carry):
                cur = keys[pl.ds(j * V, V)]
                prev_idx = jnp.maximum(j * V - 1 + lanes, 0)
                prevs = plsc.load_gather(keys, [prev_idx])
                first = jnp.logical_and(j == 0, lanes == 0)
                flag = jnp.where(jnp.logical_or(cur != prevs, first), 1, 0)
                csum = plsc.cumsum(flag)
                ranks[pl.ds(j * V, V)] = csum + carry - 1
                pos = vals[pl.ds(j * V, V)]
                orow2d[lax.div(j, CH // V),
                       pl.ds(lax.rem(j, CH // V) * V, V)] = (
                    obase + ((pos >> 3) << 4) + (pos & 7))
                return carry + jnp.max(csum)

            @pl.when(half == 0)
            def _():
                lastk = plsc.bitcast(keys[pl.ds(M - V, V)], jnp.int32)
                lk = jnp.sum(jnp.where(lanes == V - 1, lastk, 0))
                metabuf[...] = jnp.where(
                    lanes == 0, tme, jnp.where(lanes == 1, lk, 0))
                pltpu.sync_copy(metabuf, smeta.at[s])

        cpe.wait()
        plsc.subcore_barrier()

        @pl.when(jnp.logical_and(u < B, half == 1))
        def _():
            pltpu.sync_copy(smeta.at[s - 1], metabuf)
            mv = metabuf[...]
            t0 = jnp.sum(jnp.where(lanes == 0, mv, 0))
            lk = jnp.sum(jnp.where(lanes == 1, mv, 0))
            k0 = plsc.bitcast(keys[pl.ds(0, V)], jnp.int32)
            fk = jnp.sum(jnp.where(lanes == 0, k0, 0))
            adj = t0 + jnp.where(fk == lk, -1, 0)

            @plsc.parallel_loop(0, NVh, unroll=2)
            def _(j):
                ranks[pl.ds(j * V, V)] = ranks[pl.ds(j * V, V)] + adj

        @pl.when(u < B)
        def _():
            bufs = (rowbuf, rowbuf2)
            wsems = (wsem0, wsem1)

            @pl.loop(0, NCH, step=2)
            def _(t):
                for par in (0, 1):
                    tt = t + par
                    buf = bufs[par]
                    wsem = wsems[par]

                    @pl.when(tt >= 2)
                    def _():
                        pltpu.make_async_copy(
                            buf, out_hbm.at[orow2d.at[par]], wsem).wait()

                    pltpu.async_copy(
                        spe.at[ranks.at[pl.ds(tt * CH, CH)]], buf, sem
                    ).wait()
                    pltpu.async_copy(buf, out_hbm.at[orow2d.at[tt]], wsem)

            for par in (0, 1):
                pltpu.make_async_copy(
                    bufs[par], out_hbm.at[orow2d.at[par]], wsems[par]).wait()

    cp = pltpu.CompilerParams()
    if "needs_layout_passes" in pltpu.CompilerParams.__dataclass_fields__:
        cp = dataclasses.replace(cp, needs_layout_passes=False)
    if "use_tc_tiling_on_sc" in pltpu.CompilerParams.__dataclass_fields__:
        cp = dataclasses.replace(cp, use_tc_tiling_on_sc=False)

    kern = pl.kernel(
        body,
        compiler_params=cp,
        out_type=jax.ShapeDtypeStruct((B * 2 * N, D), jnp.float32),
        mesh=mesh,
        scratch_types=[
            pltpu.VMEM((M // 128, 2, 128), jnp.float32),
            pltpu.VMEM((M,), jnp.float32),
            pltpu.VMEM((M,), jnp.int32),
            pltpu.VMEM((M,), jnp.float32),
            pltpu.VMEM((M,), jnp.int32),
            pltpu.VMEM((M,), jnp.int32),
            pltpu.VMEM((NCH, CH), jnp.int32),
            pltpu.VMEM((CH, D), jnp.float32),
            pltpu.VMEM((CH, D), jnp.float32),
            pltpu.VMEM((V,), jnp.int32),
            pltpu.VMEM_SHARED((2 * B, M), jnp.float32),
            pltpu.VMEM_SHARED((2 * B, M), jnp.int32),
            pltpu.VMEM_SHARED((2 * B, V), jnp.int32),
            pltpu.VMEM_SHARED((N, D), jnp.float32),
            pltpu.SemaphoreType.DMA,
            pltpu.SemaphoreType.DMA,
            pltpu.SemaphoreType.DMA,
            pltpu.SemaphoreType.DMA,
        ],
    )
    return kern


def kernel(coords, pe):
    B, N, _ = coords.shape
    MAXLEN, D = pe.shape
    ct = coords.reshape(B, N // 128, 128, 2).transpose(0, 1, 3, 2)
    out2d = _build(B, N, MAXLEN, D)(ct, pe)
    out5d = out2d.reshape(B, N // 8, 2, 8, D)
    return out5d.transpose(0, 1, 3, 2, 4).reshape(B, N, 2 * D)


# speedup vs baseline: 1.5465x; 1.0486x over previous
import dataclasses
import functools

import jax
import jax.numpy as jnp
from jax import lax
from jax.experimental import pallas as pl
from jax.experimental.pallas import tpu as pltpu
from jax.experimental.pallas import tpu_sc as plsc

V = 16


@functools.lru_cache(maxsize=None)
def _build(B, N, MAXLEN, D):
    M = N // 2
    NVh = M // V
    NPAIRh = NVh // 2
    CH = 256
    NCH = M // CH
    num_phases = (NVh - 1).bit_length()

    mesh = plsc.VectorSubcoreMesh(core_axis_name="c", subcore_axis_name="s")

    def body(coords_hbm, pe_hbm, out_hbm, cbuf, keys, vals, pkeys, pvals,
             ranks, orow2d, rowbuf, rowbuf2, metabuf,
             skeys, svals, smeta, spe, sem, wsem0, wsem1, psem):
        c = lax.axis_index("c")
        s = lax.axis_index("s")
        u = s // 2
        half = s % 2
        task = c * B + u
        b = task // 2
        ch = task % 2
        lanes = lax.iota(jnp.int32, V)

        PS = N // (2 * B)
        cpe = pltpu.async_copy(
            pe_hbm.at[pl.ds(s * PS, PS)], spe.at[pl.ds(s * PS, PS)], psem)

        def cross_pass(off_fn, n_pair):
            @plsc.parallel_loop(0, n_pair, unroll=8)
            def _(t):
                lo_off, hi_off = off_fn(t)
                kL = keys[pl.ds(lo_off, V)]
                kH = keys[pl.ds(hi_off, V)]
                vL = vals[pl.ds(lo_off, V)]
                vH = vals[pl.ds(hi_off, V)]
                cmp = kL <= kH
                keys[pl.ds(lo_off, V)] = jnp.where(cmp, kL, kH)
                vals[pl.ds(lo_off, V)] = jnp.where(cmp, vL, vH)
                keys[pl.ds(hi_off, V)] = jnp.where(cmp, kH, kL)
                vals[pl.ds(hi_off, V)] = jnp.where(cmp, vH, vL)

        def split_pass(m):
            @plsc.parallel_loop(0, NPAIRh, unroll=4)
            def _(t):
                p = m // V
                i = lax.rem(t, p) if p > 1 else t * 0
                pair = lax.div(t, p) if p > 1 else t
                a = pair * (2 * m) + V * i
                bb = pair * (2 * m) + 2 * m - V * (i + 1)
                kA = keys[pl.ds(a, V)]
                vA = vals[pl.ds(a, V)]
                kB = lax.rev(keys[pl.ds(bb, V)], (0,))
                vB = lax.rev(vals[pl.ds(bb, V)], (0,))
                cmp = kA <= kB
                keys[pl.ds(a, V)] = jnp.where(cmp, kA, kB)
                vals[pl.ds(a, V)] = jnp.where(cmp, vA, vB)
                keys[pl.ds(bb, V)] = lax.rev(jnp.where(cmp, kB, kA), (0,))
                vals[pl.ds(bb, V)] = lax.rev(jnp.where(cmp, vB, vA), (0,))

        def stage_pass(d):
            q = (d // V).bit_length() - 1

            def off(t):
                i = lax.rem(t, 1 << q) if q > 0 else t * 0
                blk = lax.div(t, 1 << q) if q > 0 else t
                lo = blk * (2 * d) + V * i
                return lo, lo + d

            cross_pass(off, NPAIRh)

        def vsort_pass():
            @plsc.parallel_loop(0, NVh, unroll=4)
            def _(j):
                k = keys[pl.ds(j * V, V)]
                v = vals[pl.ds(j * V, V)]
                ks, vs = plsc.sort_key_val(k, v)
                keys[pl.ds(j * V, V)] = ks
                vals[pl.ds(j * V, V)] = vs

        def split16v_pass():
            @plsc.parallel_loop(0, NPAIRh, unroll=4)
            def _(t):
                a = t * 2 * V
                bb = a + V
                kA = keys[pl.ds(a, V)]
                vA = vals[pl.ds(a, V)]
                kB = lax.rev(keys[pl.ds(bb, V)], (0,))
                vB = lax.rev(vals[pl.ds(bb, V)], (0,))
                cmp = kA <= kB
                ks, vs = plsc.sort_key_val(
                    jnp.where(cmp, kA, kB), jnp.where(cmp, vA, vB))
                keys[pl.ds(a, V)] = ks
                vals[pl.ds(a, V)] = vs
                ks, vs = plsc.sort_key_val(
                    jnp.where(cmp, kB, kA), jnp.where(cmp, vB, vA))
                keys[pl.ds(bb, V)] = ks
                vals[pl.ds(bb, V)] = vs

        def stage16v_pass():
            @plsc.parallel_loop(0, NPAIRh, unroll=4)
            def _(t):
                lo = t * 2 * V
                hi = lo + V
                kL = keys[pl.ds(lo, V)]
                kH = keys[pl.ds(hi, V)]
                vL = vals[pl.ds(lo, V)]
                vH = vals[pl.ds(hi, V)]
                cmp = kL <= kH
                ks, vs = plsc.sort_key_val(
                    jnp.where(cmp, kL, kH), jnp.where(cmp, vL, vH))
                keys[pl.ds(lo, V)] = ks
                vals[pl.ds(lo, V)] = vs
                ks, vs = plsc.sort_key_val(
                    jnp.where(cmp, kH, kL), jnp.where(cmp, vH, vL))
                keys[pl.ds(hi, V)] = ks
                vals[pl.ds(hi, V)] = vs

        @pl.when(u < B)
        def _():
            pltpu.sync_copy(
                coords_hbm.at[b, pl.ds(half * (M // 128), M // 128)], cbuf)

            @plsc.parallel_loop(0, NVh, unroll=4)
            def _(j):
                row = j * V + lanes
                k = plsc.load_gather(
                    cbuf, [row >> 7, jnp.full((V,), ch, jnp.int32), row & 127])
                ks, vs = plsc.sort_key_val(k, row + half * M)
                keys[pl.ds(j * V, V)] = ks
                vals[pl.ds(j * V, V)] = vs

            split16v_pass()
            for p in range(1, num_phases):
                m = V << p
                split_pass(m)
                for qq in range(p - 1, 0, -1):
                    stage_pass(V << qq)
                stage16v_pass()

            cp1 = pltpu.async_copy(keys, skeys.at[s], sem)
            cp2 = pltpu.async_copy(vals, svals.at[s], sem)
            cp1.wait()
            cp2.wait()

        plsc.subcore_barrier()

        @pl.when(u < B)
        def _():
            cp1 = pltpu.async_copy(skeys.at[s ^ 1], pkeys, sem)
            cp2 = pltpu.async_copy(svals.at[s ^ 1], pvals, sem)
            cp1.wait()
            cp2.wait()
            hm = jnp.full((V,), half, jnp.int32) == 0

            @plsc.parallel_loop(0, NVh, unroll=4)
            def _(i):
                mk = keys[pl.ds(i * V, V)]
                mv = vals[pl.ds(i * V, V)]
                pk = lax.rev(pkeys[pl.ds(M - V - V * i, V)], (0,))
                pv = lax.rev(pvals[pl.ds(M - V - V * i, V)], (0,))
                c0 = mk <= pk
                c1 = pk <= mk
                cmp = jnp.where(hm, c0, c1)
                keys[pl.ds(i * V, V)] = jnp.where(cmp, mk, pk)
                vals[pl.ds(i * V, V)] = jnp.where(cmp, mv, pv)

            for qq in range(num_phases - 1, 0, -1):
                stage_pass(V << qq)
            stage16v_pass()

            obase = b * 2 * N + ch * 8

            @plsc.parallel_loop(0, NVh, unroll=2, carry=jnp.int32(0))
            def tme(j, carry):
                cur = keys[pl.ds(j * V, V)]
                prev_idx = jnp.maximum(j * V - 1 + lanes, 0)
                prevs = plsc.load_gather(keys, [prev_idx])
                first = jnp.logical_and(j == 0, lanes == 0)
                flag = jnp.where(jnp.logical_or(cur != prevs, first), 1, 0)
                csum = plsc.cumsum(flag)
                ranks[pl.ds(j * V, V)] = csum + carry - 1
                pos = vals[pl.ds(j * V, V)]
                orow2d[lax.div(j, CH // V),
                       pl.ds(lax.rem(j, CH // V) * V, V)] = (
                    obase + ((pos >> 3) << 4) + (pos & 7))
                return carry + jnp.max(csum)

            @pl.when(half == 0)
            def _():
                lastk = plsc.bitcast(keys[pl.ds(M - V, V)], jnp.int32)
                lk = jnp.sum(jnp.where(lanes == V - 1, lastk, 0))
                metabuf[...] = jnp.where(
                    lanes == 0, tme, jnp.where(lanes == 1, lk, 0))
                pltpu.sync_copy(metabuf, smeta.at[s])

        cpe.wait()
        plsc.subcore_barrier()

        @pl.when(jnp.logical_and(u < B, half == 1))
        def _():
            pltpu.sync_copy(smeta.at[s - 1], metabuf)
            mv = metabuf[...]
            t0 = jnp.sum(jnp.where(lanes == 0, mv, 0))
            lk = jnp.sum(jnp.where(lanes == 1, mv, 0))
            k0 = plsc.bitcast(keys[pl.ds(0, V)], jnp.int32)
            fk = jnp.sum(jnp.where(lanes == 0, k0, 0))
            adj = t0 + jnp.where(fk == lk, -1, 0)

            @plsc.parallel_loop(0, NVh, unroll=2)
            def _(j):
                ranks[pl.ds(j * V, V)] = ranks[pl.ds(j * V, V)] + adj

        @pl.when(u < B)
        def _():
            bufs = (rowbuf, rowbuf2)
            wsems = (wsem0, wsem1)

            @pl.loop(0, NCH, step=2)
            def _(t):
                for par in (0, 1):
                    tt = t + par
                    buf = bufs[par]
                    wsem = wsems[par]

                    @pl.when(tt >= 2)
                    def _():
                        pltpu.make_async_copy(
                            buf, out_hbm.at[orow2d.at[par]], wsem).wait()

                    pltpu.async_copy(
                        spe.at[ranks.at[pl.ds(tt * CH, CH)]], buf, sem
                    ).wait()
                    pltpu.async_copy(buf, out_hbm.at[orow2d.at[tt]], wsem)

            for par in (0, 1):
                pltpu.make_async_copy(
                    bufs[par], out_hbm.at[orow2d.at[par]], wsems[par]).wait()

    cp = pltpu.CompilerParams()
    if "needs_layout_passes" in pltpu.CompilerParams.__dataclass_fields__:
        cp = dataclasses.replace(cp, needs_layout_passes=False)
    if "use_tc_tiling_on_sc" in pltpu.CompilerParams.__dataclass_fields__:
        cp = dataclasses.replace(cp, use_tc_tiling_on_sc=False)

    kern = pl.kernel(
        body,
        compiler_params=cp,
        out_type=jax.ShapeDtypeStruct((B * 2 * N, D), jnp.float32),
        mesh=mesh,
        scratch_types=[
            pltpu.VMEM((M // 128, 2, 128), jnp.float32),
            pltpu.VMEM((M,), jnp.float32),
            pltpu.VMEM((M,), jnp.int32),
            pltpu.VMEM((M,), jnp.float32),
            pltpu.VMEM((M,), jnp.int32),
            pltpu.VMEM((M,), jnp.int32),
            pltpu.VMEM((NCH, CH), jnp.int32),
            pltpu.VMEM((CH, D), jnp.float32),
            pltpu.VMEM((CH, D), jnp.float32),
            pltpu.VMEM((V,), jnp.int32),
            pltpu.VMEM_SHARED((2 * B, M), jnp.float32),
            pltpu.VMEM_SHARED((2 * B, M), jnp.int32),
            pltpu.VMEM_SHARED((2 * B, V), jnp.int32),
            pltpu.VMEM_SHARED((N, D), jnp.float32),
            pltpu.SemaphoreType.DMA,
            pltpu.SemaphoreType.DMA,
            pltpu.SemaphoreType.DMA,
            pltpu.SemaphoreType.DMA,
        ],
    )
    return kern


def kernel(coords, pe):
    B, N, _ = coords.shape
    MAXLEN, D = pe.shape
    ct = coords.reshape(B, N // 128, 128, 2).transpose(0, 1, 3, 2)
    out2d = _build(B, N, MAXLEN, D)(ct, pe)
    out5d = out2d.reshape(B, N // 8, 2, 8, D)
    return out5d.transpose(0, 1, 3, 2, 4).reshape(B, N, 2 * D)
